# baseline (speedup 1.0000x reference)
import jax
import jax.numpy as jnp
from jax.experimental import pallas as pl
from jax.experimental.pallas import tpu as pltpu

_NCHUNKS = 4


def _dma_copy(x_hbm, o_hbm, sems):
    K = x_hbm.shape[0]
    rows = K // _NCHUNKS
    copies = [
        pltpu.make_async_copy(
            x_hbm.at[pl.ds(i * rows, rows), :],
            o_hbm.at[pl.ds(i * rows, rows), :],
            sems.at[i],
        )
        for i in range(_NCHUNKS)
    ]
    for c in copies:
        c.start()
    for c in copies:
        c.wait()


def kernel(embedding_weight):
    K, D = embedding_weight.shape
    return pl.pallas_call(
        _dma_copy,
        in_specs=[pl.BlockSpec(memory_space=pl.ANY)],
        out_specs=pl.BlockSpec(memory_space=pl.ANY),
        out_shape=jax.ShapeDtypeStruct((K, D), embedding_weight.dtype),
        scratch_shapes=[pltpu.SemaphoreType.DMA((_NCHUNKS,))],
    )(embedding_weight)


# speedup vs baseline: 20.3469x; 20.3469x over previous
import jax
import jax.numpy as jnp
from jax.experimental import pallas as pl
from jax.experimental.pallas import tpu as pltpu


def _copy_block(x_ref, o_ref):
    o_ref[...] = x_ref[...]


def kernel(embedding_weight):
    K, D = embedding_weight.shape
    BK = 512
    return pl.pallas_call(
        _copy_block,
        grid=(K // BK,),
        in_specs=[pl.BlockSpec((BK, D), lambda i: (i, 0))],
        out_specs=pl.BlockSpec((BK, D), lambda i: (i, 0)),
        out_shape=jax.ShapeDtypeStruct((K, D), embedding_weight.dtype),
        compiler_params=pltpu.CompilerParams(
            dimension_semantics=("arbitrary",)),
    )(embedding_weight)


# speedup vs baseline: 34.6324x; 1.7021x over previous
import jax
import jax.numpy as jnp
from jax.experimental import pallas as pl
from jax.experimental.pallas import tpu as pltpu


def _copy_block(x_ref, o_ref):
    o_ref[...] = x_ref[...]


def kernel(embedding_weight):
    K, D = embedding_weight.shape
    BK = 2048
    return pl.pallas_call(
        _copy_block,
        grid=(K // BK,),
        in_specs=[pl.BlockSpec((BK, D), lambda i: (i, 0))],
        out_specs=pl.BlockSpec((BK, D), lambda i: (i, 0)),
        out_shape=jax.ShapeDtypeStruct((K, D), embedding_weight.dtype),
        compiler_params=pltpu.CompilerParams(
            dimension_semantics=("arbitrary",)),
    )(embedding_weight)


# speedup vs baseline: 42.5955x; 1.2299x over previous
import jax
import jax.numpy as jnp
from jax.experimental import pallas as pl
from jax.experimental.pallas import tpu as pltpu


def _copy_block(x_ref, o_ref):
    o_ref[...] = x_ref[...]


def kernel(embedding_weight):
    K, D = embedding_weight.shape
    BK = 4096
    return pl.pallas_call(
        _copy_block,
        grid=(K // BK,),
        in_specs=[pl.BlockSpec((BK, D), lambda i: (i, 0))],
        out_specs=pl.BlockSpec((BK, D), lambda i: (i, 0)),
        out_shape=jax.ShapeDtypeStruct((K, D), embedding_weight.dtype),
        compiler_params=pltpu.CompilerParams(
            dimension_semantics=("arbitrary",)),
    )(embedding_weight)
